# baseline (speedup 1.0000x reference)
import jax
import jax.numpy as jnp
from jax.experimental import pallas as pl
from jax.experimental.pallas import tpu as pltpu

BLOCK = 1024
NRING = 3


def _gate_kernel(x_hbm, w_ref, o_ref, buf, sem):
    nblocks = x_hbm.shape[0] // BLOCK

    def copy(i, slot):
        return pltpu.make_async_copy(
            x_hbm.at[pl.ds(i * BLOCK, BLOCK), :],
            buf.at[slot],
            sem.at[slot],
        )

    for s in range(NRING):
        copy(s, s).start()

    w = w_ref[...]
    for i in range(nblocks):
        slot = i % NRING
        copy(i, slot).wait()
        logits_t = jax.lax.dot_general(
            w,
            buf[slot],
            dimension_numbers=(((1,), (1,)), ((), ())),
            preferred_element_type=jnp.float32,
            precision=jax.lax.Precision.DEFAULT,
        )
        if i + NRING < nblocks:
            copy(i + NRING, slot).start()
        o_ref[:, pl.ds(i * BLOCK, BLOCK)] = (logits_t > 0.0).astype(jnp.int32)


@jax.jit
def kernel(cls_hidden_states, gate_w):
    tokens, hidden = cls_hidden_states.shape
    num_experts = gate_w.shape[0]

    mask_t = pl.pallas_call(
        _gate_kernel,
        in_specs=[
            pl.BlockSpec(memory_space=pltpu.MemorySpace.HBM),
            pl.BlockSpec(memory_space=pltpu.MemorySpace.VMEM),
        ],
        out_specs=pl.BlockSpec(memory_space=pltpu.MemorySpace.VMEM),
        out_shape=jax.ShapeDtypeStruct((num_experts, tokens), jnp.int32),
        scratch_shapes=[
            pltpu.VMEM((NRING, BLOCK, hidden), jnp.float32),
            pltpu.SemaphoreType.DMA((NRING,)),
        ],
    )(cls_hidden_states, gate_w)
    return mask_t.T


# speedup vs baseline: 1.0621x; 1.0621x over previous
import jax
import jax.numpy as jnp
from jax.experimental import pallas as pl

TOKEN_BLOCK = 1024


def _gate_block(w_ref, x_ref, o_ref):
    logits_t = jax.lax.dot_general(
        w_ref[...],
        x_ref[...],
        dimension_numbers=(((1,), (1,)), ((), ())),
        preferred_element_type=jnp.float32,
        precision=jax.lax.Precision.DEFAULT,
    )
    o_ref[...] = (logits_t > 0.0).astype(jnp.int32)


@jax.jit
def kernel(cls_hidden_states, gate_w):
    tokens, hidden = cls_hidden_states.shape
    num_experts = gate_w.shape[0]

    grid = (tokens // TOKEN_BLOCK,)
    mask_t = pl.pallas_call(
        _gate_block,
        grid=grid,
        in_specs=[
            pl.BlockSpec((num_experts, hidden), lambda i: (0, 0)),
            pl.BlockSpec((TOKEN_BLOCK, hidden), lambda i: (i, 0)),
        ],
        out_specs=pl.BlockSpec((num_experts, TOKEN_BLOCK), lambda i: (0, i)),
        out_shape=jax.ShapeDtypeStruct((num_experts, tokens), jnp.int32),
    )(gate_w, cls_hidden_states)
    return mask_t.T
